# baseline (speedup 1.0000x reference)
import functools

import jax
import jax.numpy as jnp
from jax import lax
from jax.experimental import pallas as pl
from jax.experimental.pallas import tpu as pltpu
from jax.experimental.pallas import tpu_sc as plsc

N_ROWS = 6_400_000
IN_W = 5
OUT_W = 2
BLK_ROWS = 2000
GRP = 16


def kernel(inputs):
    info = plsc.get_sparse_core_info()
    nw = info.num_cores * info.num_subcores
    rows_per_w = N_ROWS // nw
    nblk = rows_per_w // BLK_ROWS

    mesh = plsc.VectorSubcoreMesh(core_axis_name="c", subcore_axis_name="s")

    @functools.partial(
        pl.kernel,
        mesh=mesh,
        out_type=jax.ShapeDtypeStruct((N_ROWS * OUT_W,), jnp.float32),
        scratch_types=[
            pltpu.VMEM((BLK_ROWS * IN_W,), jnp.float32),
            pltpu.VMEM((BLK_ROWS * OUT_W,), jnp.float32),
        ],
        compiler_params=pltpu.CompilerParams(needs_layout_passes=False),
    )
    def sc_run(in_hbm, out_hbm, in_v, out_v):
        wid = lax.axis_index("s") * info.num_cores + lax.axis_index("c")
        row0 = wid * rows_per_w
        lane = lax.iota(jnp.int32, GRP)
        gidx = lane * IN_W
        oidx = lane * OUT_W

        def blk_body(b, carry):
            base = row0 + b * BLK_ROWS
            pltpu.sync_copy(in_hbm.at[pl.ds(base * IN_W, BLK_ROWS * IN_W)], in_v)

            def grp_body(g, c2):
                i0 = gidx + g * (GRP * IN_W)
                a0 = plsc.load_gather(in_v, [i0])
                a1 = plsc.load_gather(in_v, [i0 + 1])
                a2 = plsc.load_gather(in_v, [i0 + 2])
                a3 = plsc.load_gather(in_v, [i0 + 3])
                a4 = plsc.load_gather(in_v, [i0 + 4])
                o0 = oidx + g * (GRP * OUT_W)
                plsc.store_scatter(out_v, [o0], a0 * a1 * a2)
                plsc.store_scatter(out_v, [o0 + 1], a3 * a4)
                return c2

            lax.fori_loop(0, BLK_ROWS // GRP, grp_body, 0)
            pltpu.sync_copy(out_v, out_hbm.at[pl.ds(base * OUT_W, BLK_ROWS * OUT_W)])
            return carry

        lax.fori_loop(0, nblk, blk_body, 0)

    out_flat = sc_run(inputs.reshape(-1))
    return out_flat.reshape(N_ROWS, OUT_W)


# speedup vs baseline: 26.8912x; 26.8912x over previous
import functools

import jax
import jax.numpy as jnp
from jax import lax
from jax.experimental import pallas as pl
from jax.experimental.pallas import tpu as pltpu
from jax.experimental.pallas import tpu_sc as plsc

N_COLS = 6_400_000
IN_W = 5
OUT_W = 2
BLK_COLS = 2048
N_BLKS = N_COLS // BLK_COLS
GRP = 16


def kernel(inputs):
    info = plsc.get_sparse_core_info()
    nw = info.num_cores * info.num_subcores

    mesh = plsc.VectorSubcoreMesh(core_axis_name="c", subcore_axis_name="s")

    @functools.partial(
        pl.kernel,
        mesh=mesh,
        out_type=jax.ShapeDtypeStruct((OUT_W, N_COLS), jnp.float32),
        scratch_types=[
            pltpu.VMEM((IN_W, BLK_COLS), jnp.float32),
            pltpu.VMEM((OUT_W, BLK_COLS), jnp.float32),
        ],
        compiler_params=pltpu.CompilerParams(needs_layout_passes=False),
    )
    def sc_run(in_hbm, out_hbm, in_v, out_v):
        wid = lax.axis_index("s") * info.num_cores + lax.axis_index("c")
        t0 = wid * N_BLKS // nw
        t1 = (wid + 1) * N_BLKS // nw

        def blk_body(b, carry):
            base = pl.multiple_of(b * BLK_COLS, BLK_COLS)
            pltpu.sync_copy(in_hbm.at[:, pl.ds(base, BLK_COLS)], in_v)

            def grp_body(g, c2):
                j = g * GRP
                a0 = in_v[0, pl.ds(j, GRP)]
                a1 = in_v[1, pl.ds(j, GRP)]
                a2 = in_v[2, pl.ds(j, GRP)]
                a3 = in_v[3, pl.ds(j, GRP)]
                a4 = in_v[4, pl.ds(j, GRP)]
                out_v[0, pl.ds(j, GRP)] = a0 * a1 * a2
                out_v[1, pl.ds(j, GRP)] = a3 * a4
                return c2

            lax.fori_loop(0, BLK_COLS // GRP, grp_body, 0)
            pltpu.sync_copy(out_v, out_hbm.at[:, pl.ds(base, BLK_COLS)])
            return carry

        lax.fori_loop(t0, t1, blk_body, 0)

    return sc_run(inputs.T).T


# speedup vs baseline: 27.3254x; 1.0161x over previous
import functools

import jax
import jax.numpy as jnp
from jax import lax
from jax.experimental import pallas as pl
from jax.experimental.pallas import tpu as pltpu
from jax.experimental.pallas import tpu_sc as plsc

N_COLS = 6_400_000
IN_W = 5
OUT_W = 2
BLK_COLS = 2048
N_BLKS = N_COLS // BLK_COLS
GRP = 16


def kernel(inputs):
    info = plsc.get_sparse_core_info()
    nw = info.num_cores * info.num_subcores

    mesh = plsc.VectorSubcoreMesh(core_axis_name="c", subcore_axis_name="s")

    @functools.partial(
        pl.kernel,
        mesh=mesh,
        out_type=jax.ShapeDtypeStruct((OUT_W, N_COLS), jnp.float32),
        scratch_types=[
            pltpu.VMEM((IN_W, BLK_COLS), jnp.float32),
            pltpu.VMEM((OUT_W, BLK_COLS), jnp.float32),
        ],
        compiler_params=pltpu.CompilerParams(needs_layout_passes=False),
    )
    def sc_run(in_hbm, out_hbm, in_v, out_v):
        wid = lax.axis_index("s") * info.num_cores + lax.axis_index("c")
        t0 = wid * N_BLKS // nw
        t1 = (wid + 1) * N_BLKS // nw

        def blk_body(b, carry):
            base = pl.multiple_of(b * BLK_COLS, BLK_COLS)
            pltpu.sync_copy(in_hbm.at[:, pl.ds(base, BLK_COLS)], in_v)

            for g in range(BLK_COLS // GRP):
                j = g * GRP
                a0 = in_v[0, pl.ds(j, GRP)]
                a1 = in_v[1, pl.ds(j, GRP)]
                a2 = in_v[2, pl.ds(j, GRP)]
                a3 = in_v[3, pl.ds(j, GRP)]
                a4 = in_v[4, pl.ds(j, GRP)]
                out_v[0, pl.ds(j, GRP)] = a0 * a1 * a2
                out_v[1, pl.ds(j, GRP)] = a3 * a4
            pltpu.sync_copy(out_v, out_hbm.at[:, pl.ds(base, BLK_COLS)])
            return carry

        lax.fori_loop(t0, t1, blk_body, 0)

    return sc_run(inputs.T).T


# speedup vs baseline: 39.7511x; 1.4547x over previous
import functools

import jax
import jax.numpy as jnp
from jax import lax
from jax.experimental import pallas as pl
from jax.experimental.pallas import tpu as pltpu
from jax.experimental.pallas import tpu_sc as plsc

N_COLS = 6_400_000
IN_W = 5
OUT_W = 2
BLK_COLS = 2048
N_BLKS = N_COLS // BLK_COLS
GRP = 16
NBUF = 2


def kernel(inputs):
    info = plsc.get_sparse_core_info()
    nw = info.num_cores * info.num_subcores
    nsteps = -(-N_BLKS // nw)
    npairs = nsteps // NBUF

    mesh = plsc.VectorSubcoreMesh(core_axis_name="c", subcore_axis_name="s")

    @functools.partial(
        pl.kernel,
        mesh=mesh,
        out_type=jax.ShapeDtypeStruct((OUT_W, N_COLS), jnp.float32),
        scratch_types=[
            pltpu.VMEM((NBUF, IN_W, BLK_COLS), jnp.float32),
            pltpu.VMEM((NBUF, OUT_W, BLK_COLS), jnp.float32),
            [pltpu.SemaphoreType.DMA] * NBUF,
            [pltpu.SemaphoreType.DMA] * NBUF,
        ],
        compiler_params=pltpu.CompilerParams(needs_layout_passes=False),
    )
    def sc_run(in_hbm, out_hbm, in_v, out_v, in_sems, out_sems):
        wid = lax.axis_index("s") * info.num_cores + lax.axis_index("c")
        t0 = wid * N_BLKS // nw

        def in_slab(t):
            base = pl.multiple_of(t * BLK_COLS, BLK_COLS)
            return in_hbm.at[:, pl.ds(base, BLK_COLS)]

        def out_slab(t):
            base = pl.multiple_of(t * BLK_COLS, BLK_COLS)
            return out_hbm.at[:, pl.ds(base, BLK_COLS)]

        def blk_t(k):
            return jnp.minimum(t0 + k, N_BLKS - 1)

        for bi in range(NBUF):
            pltpu.async_copy(in_slab(blk_t(bi)), in_v.at[bi], in_sems[bi])

        def pair_body(k, carry):
            for bi in range(NBUF):
                t = blk_t(NBUF * k + bi)
                pltpu.make_async_copy(in_slab(t), in_v.at[bi], in_sems[bi]).wait()

                @pl.when(k >= 1)
                def _():
                    pltpu.make_async_copy(
                        out_v.at[bi], out_slab(t), out_sems[bi]
                    ).wait()

                for g in range(BLK_COLS // GRP):
                    j = g * GRP
                    a0 = in_v[bi, 0, pl.ds(j, GRP)]
                    a1 = in_v[bi, 1, pl.ds(j, GRP)]
                    a2 = in_v[bi, 2, pl.ds(j, GRP)]
                    a3 = in_v[bi, 3, pl.ds(j, GRP)]
                    a4 = in_v[bi, 4, pl.ds(j, GRP)]
                    out_v[bi, 0, pl.ds(j, GRP)] = a0 * a1 * a2
                    out_v[bi, 1, pl.ds(j, GRP)] = a3 * a4

                pltpu.async_copy(out_v.at[bi], out_slab(t), out_sems[bi])

                @pl.when(k < npairs - 1)
                def _():
                    tn = blk_t(NBUF * (k + 1) + bi)
                    pltpu.async_copy(in_slab(tn), in_v.at[bi], in_sems[bi])

            return carry

        lax.fori_loop(0, npairs, pair_body, 0)
        for bi in range(NBUF):
            t = blk_t(nsteps - NBUF + bi)
            pltpu.make_async_copy(out_v.at[bi], out_slab(t), out_sems[bi]).wait()

    return sc_run(inputs.T).T


# speedup vs baseline: 40.6762x; 1.0233x over previous
import functools

import jax
import jax.numpy as jnp
from jax import lax
from jax.experimental import pallas as pl
from jax.experimental.pallas import tpu as pltpu
from jax.experimental.pallas import tpu_sc as plsc

N_COLS = 6_400_000
IN_W = 5
OUT_W = 2
BLK_COLS = 2048
N_BLKS = N_COLS // BLK_COLS
GRP = 16
NBUF = 3


def kernel(inputs):
    info = plsc.get_sparse_core_info()
    nw = info.num_cores * info.num_subcores
    nsteps = -(-(-(-N_BLKS // nw)) // NBUF) * NBUF
    npairs = nsteps // NBUF

    mesh = plsc.VectorSubcoreMesh(core_axis_name="c", subcore_axis_name="s")

    @functools.partial(
        pl.kernel,
        mesh=mesh,
        out_type=jax.ShapeDtypeStruct((OUT_W, N_COLS), jnp.float32),
        scratch_types=[
            pltpu.VMEM((NBUF, IN_W, BLK_COLS), jnp.float32),
            pltpu.VMEM((NBUF, OUT_W, BLK_COLS), jnp.float32),
            [pltpu.SemaphoreType.DMA] * NBUF,
            [pltpu.SemaphoreType.DMA] * NBUF,
        ],
        compiler_params=pltpu.CompilerParams(needs_layout_passes=False),
    )
    def sc_run(in_hbm, out_hbm, in_v, out_v, in_sems, out_sems):
        wid = lax.axis_index("s") * info.num_cores + lax.axis_index("c")
        t0 = wid * N_BLKS // nw

        def in_slab(t):
            base = pl.multiple_of(t * BLK_COLS, BLK_COLS)
            return in_hbm.at[:, pl.ds(base, BLK_COLS)]

        def out_slab(t):
            base = pl.multiple_of(t * BLK_COLS, BLK_COLS)
            return out_hbm.at[:, pl.ds(base, BLK_COLS)]

        def blk_t(k):
            return jnp.minimum(t0 + k, N_BLKS - 1)

        for bi in range(NBUF):
            pltpu.async_copy(in_slab(blk_t(bi)), in_v.at[bi], in_sems[bi])

        def pair_body(k, carry):
            for bi in range(NBUF):
                t = blk_t(NBUF * k + bi)
                pltpu.make_async_copy(in_slab(t), in_v.at[bi], in_sems[bi]).wait()

                @pl.when(k >= 1)
                def _():
                    pltpu.make_async_copy(
                        out_v.at[bi], out_slab(t), out_sems[bi]
                    ).wait()

                for g in range(BLK_COLS // GRP):
                    j = g * GRP
                    a0 = in_v[bi, 0, pl.ds(j, GRP)]
                    a1 = in_v[bi, 1, pl.ds(j, GRP)]
                    a2 = in_v[bi, 2, pl.ds(j, GRP)]
                    a3 = in_v[bi, 3, pl.ds(j, GRP)]
                    a4 = in_v[bi, 4, pl.ds(j, GRP)]
                    out_v[bi, 0, pl.ds(j, GRP)] = a0 * a1 * a2
                    out_v[bi, 1, pl.ds(j, GRP)] = a3 * a4

                pltpu.async_copy(out_v.at[bi], out_slab(t), out_sems[bi])

                @pl.when(k < npairs - 1)
                def _():
                    tn = blk_t(NBUF * (k + 1) + bi)
                    pltpu.async_copy(in_slab(tn), in_v.at[bi], in_sems[bi])

            return carry

        lax.fori_loop(0, npairs, pair_body, 0)
        for bi in range(NBUF):
            t = blk_t(nsteps - NBUF + bi)
            pltpu.make_async_copy(out_v.at[bi], out_slab(t), out_sems[bi]).wait()

    return sc_run(inputs.T).T
